# baseline (speedup 1.0000x reference)
import functools

import jax
import jax.numpy as jnp
from jax import lax
from jax.experimental import pallas as pl
from jax.experimental.pallas import tpu as pltpu
from jax.experimental.pallas import tpu_sc as plsc

N = 16384
D_DENSE = 2048
D_SPARSE = 1024
E = 64
TOP_K = 8
BN = 512

_NC, _NS, _L = 2, 16, 16
_NW = _NC * _NS
_ROWS = N // _NW


def _tc_body(d_ref, s_ref, wd_ref, bd_ref, ws_ref, bs_ref, b_ref,
             logits_ref, cd_ref, cs_ref):
    cd = jnp.dot(d_ref[...], wd_ref[...],
                 preferred_element_type=jnp.float32) + bd_ref[...]
    cs = jnp.dot(s_ref[...], ws_ref[...],
                 preferred_element_type=jnp.float32) + bs_ref[...]
    cd_ref[...] = cd
    cs_ref[...] = cs
    logits_ref[...] = b_ref[...] + cd + cs


def _sc_body(logits_hbm, mask_hbm, w_hbm, lg_v, mk_v, wt_v):
    wid = lax.axis_index("s") * _NC + lax.axis_index("c")
    base = wid * _ROWS * E
    pltpu.sync_copy(logits_hbm.at[pl.ds(base, _ROWS * E)], lg_v)

    lane = lax.iota(jnp.int32, _L)
    perm_hi = (lane + 8) & 15

    def merge_top8(a, b):
        b_perm = b.at[perm_hi].get(mode="promise_in_bounds")
        comb = jnp.where(lane < 8, a, b_perm)
        s, _ = plsc.sort_key_val(comb, lane, descending=True)
        return s

    def tok(t, carry):
        v = [lg_v[pl.ds(t * E + _L * i, _L)] for i in range(4)]
        s = [plsc.sort_key_val(v[i], lane, descending=True)[0]
             for i in range(4)]
        fin = merge_top8(merge_top8(s[0], s[1]), merge_top8(s[2], s[3]))
        t8 = fin.at[lane * 0 + (TOP_K - 1)].get(mode="promise_in_bounds")
        m0 = fin.at[lane * 0].get(mode="promise_in_bounds")
        es = []
        for i in range(4):
            ge = v[i] >= t8
            mk_v[pl.ds(t * E + _L * i, _L)] = jnp.where(ge, 1.0, 0.0)
            es.append(jnp.where(ge, jnp.exp(v[i] - m0), 0.0))
        tot = jnp.sum(es[0] + es[1] + es[2] + es[3])
        for i in range(4):
            wt_v[pl.ds(t * E + _L * i, _L)] = es[i] / tot
        return carry

    lax.fori_loop(0, _ROWS, tok, 0)
    pltpu.sync_copy(mk_v, mask_hbm.at[pl.ds(base, _ROWS * E)])
    pltpu.sync_copy(wt_v, w_hbm.at[pl.ds(base, _ROWS * E)])


def _sc_route(logits):
    f = pl.kernel(
        _sc_body,
        out_type=[jax.ShapeDtypeStruct((N * E,), jnp.float32)] * 2,
        mesh=plsc.VectorSubcoreMesh(
            core_axis_name="c", subcore_axis_name="s",
            num_cores=_NC, num_subcores=_NS),
        scratch_types=[pltpu.VMEM((_ROWS * E,), jnp.float32)] * 3,
        compiler_params=pltpu.CompilerParams(needs_layout_passes=False),
    )
    mask, w = f(logits.reshape(N * E))
    return mask.reshape(N, E), w.reshape(N, E)


@jax.jit
def _router(dense, sparse, W_dense, b_dense, W_sparse, b_sparse, bias):
    grid = (N // BN,)
    row_spec = pl.BlockSpec((BN, E), lambda i: (i, 0))
    full = lambda shape: pl.BlockSpec(shape, lambda i: (0, 0))
    logits, cd, cs = pl.pallas_call(
        _tc_body,
        grid=grid,
        in_specs=[
            pl.BlockSpec((BN, D_DENSE), lambda i: (i, 0)),
            pl.BlockSpec((BN, D_SPARSE), lambda i: (i, 0)),
            full((D_DENSE, E)),
            full((1, E)),
            full((D_SPARSE, E)),
            full((1, E)),
            full((1, E)),
        ],
        out_specs=[row_spec] * 3,
        out_shape=[jax.ShapeDtypeStruct((N, E), jnp.float32)] * 3,
    )(dense, sparse, W_dense, b_dense.reshape(1, E),
      W_sparse, b_sparse.reshape(1, E), bias.reshape(1, E))
    mask, weights = _sc_route(logits)
    return logits, weights, mask, cd, cs


def kernel(dense, sparse, W_dense, b_dense, W_sparse, b_sparse, bias):
    logits, weights, topk_mask, c_dense, c_sparse = _router(
        dense, sparse, W_dense, b_dense, W_sparse, b_sparse, bias)
    return (logits, weights, topk_mask, c_dense, c_sparse)


# speedup vs baseline: 1.0230x; 1.0230x over previous
import functools

import jax
import jax.numpy as jnp
from jax import lax
from jax.experimental import pallas as pl
from jax.experimental.pallas import tpu as pltpu
from jax.experimental.pallas import tpu_sc as plsc

N = 16384
D_DENSE = 2048
D_SPARSE = 1024
E = 64
TOP_K = 8
BN = 1024

_NC, _NS, _L = 2, 16, 16
_NW = _NC * _NS
_ROWS = N // _NW


def _tc_body(d_ref, s_ref, wd_ref, bd_ref, ws_ref, bs_ref, b_ref,
             logits_ref, cd_ref, cs_ref):
    cd = jnp.dot(d_ref[...], wd_ref[...],
                 preferred_element_type=jnp.float32) + bd_ref[...]
    cs = jnp.dot(s_ref[...], ws_ref[...],
                 preferred_element_type=jnp.float32) + bs_ref[...]
    cd_ref[...] = cd
    cs_ref[...] = cs
    logits_ref[...] = b_ref[...] + cd + cs


def _sc_body(logits_hbm, mask_hbm, w_hbm, lg_v, mk_v, wt_v):
    wid = lax.axis_index("s") * _NC + lax.axis_index("c")
    base = wid * _ROWS * E
    pltpu.sync_copy(logits_hbm.at[pl.ds(base, _ROWS * E)], lg_v)

    lane = lax.iota(jnp.int32, _L)
    perm_hi = (lane + 8) & 15

    def merge_top8(a, b):
        b_perm = b.at[perm_hi].get(mode="promise_in_bounds")
        comb = jnp.where(lane < 8, a, b_perm)
        s, _ = plsc.sort_key_val(comb, lane, descending=True)
        return s

    def tok(t, carry):
        v = [lg_v[pl.ds(t * E + _L * i, _L)] for i in range(4)]
        s = [plsc.sort_key_val(v[i], lane, descending=True)[0]
             for i in range(4)]
        fin = merge_top8(merge_top8(s[0], s[1]), merge_top8(s[2], s[3]))
        t8 = fin.at[lane * 0 + (TOP_K - 1)].get(mode="promise_in_bounds")
        m0 = fin.at[lane * 0].get(mode="promise_in_bounds")
        es = []
        for i in range(4):
            ge = v[i] >= t8
            mk_v[pl.ds(t * E + _L * i, _L)] = jnp.where(ge, 1.0, 0.0)
            es.append(jnp.where(ge, jnp.exp(v[i] - m0), 0.0))
        tot = jnp.sum(es[0] + es[1] + es[2] + es[3])
        for i in range(4):
            wt_v[pl.ds(t * E + _L * i, _L)] = es[i] / tot
        return carry

    lax.fori_loop(0, _ROWS, tok, 0)
    pltpu.sync_copy(mk_v, mask_hbm.at[pl.ds(base, _ROWS * E)])
    pltpu.sync_copy(wt_v, w_hbm.at[pl.ds(base, _ROWS * E)])


def _sc_route(logits):
    f = pl.kernel(
        _sc_body,
        out_type=[jax.ShapeDtypeStruct((N * E,), jnp.float32)] * 2,
        mesh=plsc.VectorSubcoreMesh(
            core_axis_name="c", subcore_axis_name="s",
            num_cores=_NC, num_subcores=_NS),
        scratch_types=[pltpu.VMEM((_ROWS * E,), jnp.float32)] * 3,
        compiler_params=pltpu.CompilerParams(needs_layout_passes=False),
    )
    mask, w = f(logits.reshape(N * E))
    return mask.reshape(N, E), w.reshape(N, E)


@jax.jit
def _router(dense, sparse, W_dense, b_dense, W_sparse, b_sparse, bias):
    grid = (N // BN,)
    row_spec = pl.BlockSpec((BN, E), lambda i: (i, 0))
    full = lambda shape: pl.BlockSpec(shape, lambda i: (0, 0))
    logits, cd, cs = pl.pallas_call(
        _tc_body,
        grid=grid,
        in_specs=[
            pl.BlockSpec((BN, D_DENSE), lambda i: (i, 0)),
            pl.BlockSpec((BN, D_SPARSE), lambda i: (i, 0)),
            full((D_DENSE, E)),
            full((1, E)),
            full((D_SPARSE, E)),
            full((1, E)),
            full((1, E)),
        ],
        out_specs=[row_spec] * 3,
        out_shape=[jax.ShapeDtypeStruct((N, E), jnp.float32)] * 3,
    )(dense, sparse, W_dense, b_dense.reshape(1, E),
      W_sparse, b_sparse.reshape(1, E), bias.reshape(1, E))
    mask, weights = _sc_route(logits)
    return logits, weights, mask, cd, cs


def kernel(dense, sparse, W_dense, b_dense, W_sparse, b_sparse, bias):
    logits, weights, topk_mask, c_dense, c_sparse = _router(
        dense, sparse, W_dense, b_dense, W_sparse, b_sparse, bias)
    return (logits, weights, topk_mask, c_dense, c_sparse)
